# baseline (speedup 1.0000x reference)
import jax
import jax.numpy as jnp
from jax.experimental import pallas as pl
from jax.experimental.pallas import tpu as pltpu

_N = 4096
_LATENT = 2048
_SYM = 128
_K = 1024
_TOPK = 128
_BN = 1024
_BISECT_ITERS = 18


def _vq_mean_kernel(z_ref, w_ref, b_ref, cb_ref, out_ref):
    z = z_ref[...]
    w = w_ref[...]
    b = b_ref[...]
    cb = cb_ref[...]

    hp = jax.lax.Precision.HIGHEST
    zp = jax.lax.dot_general(z, w, (((1,), (0,)), ((), ())),
                             preferred_element_type=jnp.float32) + b
    ones = jnp.ones((1, _SYM), dtype=jnp.float32)
    cn = jax.lax.dot_general(ones, cb * cb, (((1,), (1,)), ((), ())),
                             precision=hp,
                             preferred_element_type=jnp.float32)
    dot = jax.lax.dot_general(zp.astype(jnp.bfloat16), cb.astype(jnp.bfloat16),
                              (((1,), (1,)), ((), ())),
                              preferred_element_type=jnp.float32)
    s = cn - 2.0 * dot

    lo0 = jnp.min(s, axis=1, keepdims=True) - 1.0
    hi0 = jnp.max(s, axis=1, keepdims=True)

    def _bisect(_, carry):
        lo, hi = carry
        mid = 0.5 * (lo + hi)
        c = jnp.sum((s <= mid).astype(jnp.float32), axis=1, keepdims=True)
        pred = c >= _TOPK
        return jnp.where(pred, lo, mid), jnp.where(pred, mid, hi)

    lo, hi = jax.lax.fori_loop(0, _BISECT_ITERS, _bisect, (lo0, hi0))

    lt = (s <= lo).astype(jnp.float32)
    grp = ((s > lo) & (s <= hi)).astype(jnp.float32)
    c_lo = jnp.sum(lt, axis=1, keepdims=True)
    g = jnp.maximum(jnp.sum(grp, axis=1, keepdims=True), 1.0)
    wts = lt + grp * ((_TOPK - c_lo) / g)

    w_hi = wts.astype(jnp.bfloat16)
    w_lo = (wts - w_hi.astype(jnp.float32)).astype(jnp.bfloat16)
    c_hi = cb.astype(jnp.bfloat16)
    c_lo2 = (cb - c_hi.astype(jnp.float32)).astype(jnp.bfloat16)
    dn = (((1,), (0,)), ((), ()))
    acc = jax.lax.dot_general(w_hi, c_hi, dn, preferred_element_type=jnp.float32)
    acc += jax.lax.dot_general(w_hi, c_lo2, dn, preferred_element_type=jnp.float32)
    acc += jax.lax.dot_general(w_lo, c_hi, dn, preferred_element_type=jnp.float32)
    out_ref[...] = acc * (1.0 / _TOPK)


def kernel(z, codebook, W, b):
    zb = z.astype(jnp.bfloat16)
    wt = W.T.astype(jnp.bfloat16)
    b2 = b.reshape(1, _SYM)
    grid = (_N // _BN,)
    return pl.pallas_call(
        _vq_mean_kernel,
        grid=grid,
        in_specs=[
            pl.BlockSpec((_BN, _LATENT), lambda i: (i, 0)),
            pl.BlockSpec((_LATENT, _SYM), lambda i: (0, 0)),
            pl.BlockSpec((1, _SYM), lambda i: (0, 0)),
            pl.BlockSpec((_K, _SYM), lambda i: (0, 0)),
        ],
        out_specs=pl.BlockSpec((_BN, _SYM), lambda i: (i, 0)),
        out_shape=jax.ShapeDtypeStruct((_N, _SYM), jnp.float32),
        compiler_params=pltpu.CompilerParams(
            dimension_semantics=("parallel",),
        ),
    )(zb, wt, b2, codebook)


# speedup vs baseline: 1.2146x; 1.2146x over previous
import jax
import jax.numpy as jnp
from jax.experimental import pallas as pl
from jax.experimental.pallas import tpu as pltpu

_N = 4096
_LATENT = 2048
_SYM = 128
_K = 1024
_TOPK = 128
_BN = 1024
_BISECT_ITERS = 18


def _vq_mean_kernel(z_ref, w_ref, b_ref, cb_ref, out_ref):
    z = z_ref[...]
    w = w_ref[...]
    b = b_ref[...]
    cb = cb_ref[...]

    hp = jax.lax.Precision.HIGHEST
    zp = jax.lax.dot_general(z.astype(jnp.bfloat16), w.astype(jnp.bfloat16),
                             (((1,), (1,)), ((), ())),
                             preferred_element_type=jnp.float32) + b
    ones = jnp.ones((1, _SYM), dtype=jnp.float32)
    cn = jax.lax.dot_general(ones, cb * cb, (((1,), (1,)), ((), ())),
                             precision=hp,
                             preferred_element_type=jnp.float32)
    dot = jax.lax.dot_general(zp.astype(jnp.bfloat16), cb.astype(jnp.bfloat16),
                              (((1,), (1,)), ((), ())),
                              preferred_element_type=jnp.float32)
    s = cn - 2.0 * dot

    lo0 = jnp.min(s, axis=1, keepdims=True) - 1.0
    hi0 = jnp.max(s, axis=1, keepdims=True)

    def _bisect(_, carry):
        lo, hi = carry
        mid = 0.5 * (lo + hi)
        c = jnp.sum((s <= mid).astype(jnp.float32), axis=1, keepdims=True)
        pred = c >= _TOPK
        return jnp.where(pred, lo, mid), jnp.where(pred, mid, hi)

    lo, hi = jax.lax.fori_loop(0, _BISECT_ITERS, _bisect, (lo0, hi0))

    lt = (s <= lo).astype(jnp.float32)
    grp = ((s > lo) & (s <= hi)).astype(jnp.float32)
    c_lo = jnp.sum(lt, axis=1, keepdims=True)
    g = jnp.maximum(jnp.sum(grp, axis=1, keepdims=True), 1.0)
    wts = lt + grp * ((_TOPK - c_lo) / g)

    w_hi = wts.astype(jnp.bfloat16)
    w_lo = (wts - w_hi.astype(jnp.float32)).astype(jnp.bfloat16)
    c_hi = cb.astype(jnp.bfloat16)
    c_lo2 = (cb - c_hi.astype(jnp.float32)).astype(jnp.bfloat16)
    dn = (((1,), (0,)), ((), ()))
    acc = jax.lax.dot_general(w_hi, c_hi, dn, preferred_element_type=jnp.float32)
    acc += jax.lax.dot_general(w_hi, c_lo2, dn, preferred_element_type=jnp.float32)
    acc += jax.lax.dot_general(w_lo, c_hi, dn, preferred_element_type=jnp.float32)
    out_ref[...] = acc * (1.0 / _TOPK)


def kernel(z, codebook, W, b):
    b2 = b.reshape(1, _SYM)
    grid = (_N // _BN,)
    return pl.pallas_call(
        _vq_mean_kernel,
        grid=grid,
        in_specs=[
            pl.BlockSpec((_BN, _LATENT), lambda i: (i, 0)),
            pl.BlockSpec((_SYM, _LATENT), lambda i: (0, 0)),
            pl.BlockSpec((1, _SYM), lambda i: (0, 0)),
            pl.BlockSpec((_K, _SYM), lambda i: (0, 0)),
        ],
        out_specs=pl.BlockSpec((_BN, _SYM), lambda i: (i, 0)),
        out_shape=jax.ShapeDtypeStruct((_N, _SYM), jnp.float32),
        compiler_params=pltpu.CompilerParams(
            dimension_semantics=("parallel",),
        ),
    )(z, W, b2, codebook)
